# baseline (speedup 1.0000x reference)
import functools

import jax
import jax.numpy as jnp
from jax import lax
from jax.experimental import pallas as pl
from jax.experimental.pallas import tpu as pltpu
from jax.experimental.pallas import tpu_sc as plsc

N_NODES = 10000
N_EDGES = 160000
DIM = 256
NF = DIM // 16

N_BLK = 1024
NP = 10240
HALF = NP // 2
NODES_PER_TILE = HALF // 16

STAGE = 1024
N_STAGES = 10
E_PER_TILE = STAGE * N_STAGES
EP = 16 * E_PER_TILE
SUP = 512
CAP = SUP
CHUNK = 64
IDXROWS = CAP // CHUNK


def _mm_body(x_ref, w_ref, attm_ref, h_ref, ab_ref):
    h = jnp.dot(x_ref[...], w_ref[...], preferred_element_type=jnp.float32)
    h_ref[...] = h
    ab_ref[...] = jnp.dot(h, attm_ref[...], preferred_element_type=jnp.float32)


def _project(x, W, attm):
    xp = jnp.pad(x, ((0, NP - N_NODES), (0, 0)))
    grid = NP // N_BLK
    return pl.pallas_call(
        _mm_body,
        grid=(grid,),
        in_specs=[
            pl.BlockSpec((N_BLK, DIM), lambda i: (i, 0)),
            pl.BlockSpec((DIM, DIM), lambda i: (0, 0)),
            pl.BlockSpec((DIM, 8), lambda i: (0, 0)),
        ],
        out_specs=[
            pl.BlockSpec((N_BLK, DIM), lambda i: (i, 0)),
            pl.BlockSpec((N_BLK, 8), lambda i: (i, 0)),
        ],
        out_shape=[
            jax.ShapeDtypeStruct((NP, DIM), jnp.float32),
            jax.ShapeDtypeStruct((NP, 8), jnp.float32),
        ],
    )(xp, W, attm)


def _sc_body(h_hbm, a_hbm, b_hbm, src2d, dst2d, out_hbm,
             wtab_v, sstage_v, dstage_v, slist_v, dlist_v, wlist_v,
             didx_v, rows_v, half_v, recip_v, agg_s, denom_s, sem0):
    cid = lax.axis_index("c")
    sid = lax.axis_index("s")
    lo = cid * HALF
    zeros16 = jnp.zeros((16,), jnp.float32)
    izeros16 = jnp.zeros((16,), jnp.int32)
    iota16 = jnp.arange(16, dtype=jnp.int32)

    def stage_idx(t):
        base_row = sid * (E_PER_TILE // 128) + t * (STAGE // 128)
        pltpu.sync_copy(src2d.at[pl.ds(base_row, STAGE // 128)], sstage_v)
        pltpu.sync_copy(dst2d.at[pl.ds(base_row, STAGE // 128)], dstage_v)

    def slot_get(i):
        return rows_v[i >> 4, pl.ds((i & 15) * 16, 16)]

    def slot_set(i, val):
        rows_v[i >> 4, pl.ds((i & 15) * 16, 16)] = val

    pltpu.sync_copy(a_hbm, wtab_v)

    def prea(t, carry):
        stage_idx(t)

        def prea_j(j, carry2):
            s16 = sstage_v[j >> 3, pl.ds((j & 7) * 16, 16)]
            slot_set(t * (STAGE // 16) + j, plsc.load_gather(wtab_v, [s16]))
            return carry2
        lax.fori_loop(0, STAGE // 16, prea_j, 0)
        return carry
    lax.fori_loop(0, N_STAGES, prea, 0)

    pltpu.sync_copy(b_hbm, wtab_v)

    def preb(t, carry):
        stage_idx(t)

        def preb_j(j, carry2):
            i = t * (STAGE // 16) + j
            d16 = dstage_v[j >> 3, pl.ds((j & 7) * 16, 16)]
            z = slot_get(i) + plsc.load_gather(wtab_v, [d16])
            z = jnp.where(z >= 0.0, z, 0.2 * z)
            wv = jnp.exp(z)
            gid = sid * E_PER_TILE + i * 16 + iota16
            slot_set(i, jnp.where(gid < N_EDGES, wv, 0.0))
            return carry2
        lax.fori_loop(0, STAGE // 16, preb_j, 0)
        return carry
    lax.fori_loop(0, N_STAGES, preb, 0)

    def wcopy(i, carry):
        wtab_v[pl.ds(i * 16, 16)] = slot_get(i)
        return carry
    lax.fori_loop(0, E_PER_TILE // 16, wcopy, 0)

    def zero_lists(i, carry):
        slist_v[pl.ds(i * 16, 16)] = izeros16
        dlist_v[pl.ds(i * 16, 16)] = izeros16
        wlist_v[pl.ds(i * 16, 16)] = zeros16
        return carry
    lax.fori_loop(0, CAP // 16, zero_lists, 0)

    def zero_half(r, carry):
        for f in range(8):
            half_v[r, pl.ds(f * 16, 16)] = zeros16
        return carry
    lax.fori_loop(0, CHUNK, zero_half, 0)
    for j in range(NODES_PER_TILE // CHUNK + 1):
        rb = j * CHUNK
        if rb < NODES_PER_TILE:
            n = min(CHUNK, NODES_PER_TILE - rb)
            for f in range(2):
                pltpu.sync_copy(
                    half_v.at[pl.ds(0, n)],
                    agg_s.at[f, pl.ds(sid * NODES_PER_TILE + rb, n)])

    def zero_recip(i, carry):
        recip_v[pl.ds(i * 16, 16)] = zeros16
        return carry
    lax.fori_loop(0, NODES_PER_TILE // 16, zero_recip, 0)
    pltpu.sync_copy(recip_v, denom_s.at[pl.ds(sid * NODES_PER_TILE,
                                              NODES_PER_TILE)])
    plsc.subcore_barrier()

    def stage_body(t, carry):
        stage_idx(t)

        def sup_body(u, carry1):
            def fvec(j, nt):
                jj = u * (SUP // 16) + j
                s16 = sstage_v[jj >> 3, pl.ds((jj & 7) * 16, 16)]
                d16 = dstage_v[jj >> 3, pl.ds((jj & 7) * 16, 16)]
                ei = t * STAGE + u * SUP + j * 16
                w16 = wtab_v[pl.ds(ei, 16)]
                keep = (d16 >= lo) & (d16 < lo + HALF) & (w16 > 0.0)
                plsc.store_compressed(slist_v.at[pl.ds(nt, 16)], s16,
                                      mask=keep)
                plsc.store_compressed(dlist_v.at[pl.ds(nt, 16)], d16 - lo,
                                      mask=keep)
                plsc.store_compressed(wlist_v.at[pl.ds(nt, 16)], w16,
                                      mask=keep)
                return nt + plsc.all_reduce_population_count(keep)[0]
            nt = lax.fori_loop(0, SUP // 16, fvec, 0)

            def tails(j, carry2):
                lane = j * 16 + iota16
                wv = wlist_v[pl.ds(j * 16, 16)]
                wlist_v[pl.ds(j * 16, 16)] = jnp.where(lane < nt, wv, 0.0)
                didx_v[j >> 2, pl.ds((j & 3) * 16, 16)] = (
                    dlist_v[pl.ds(j * 16, 16)])
                return carry2
            lax.fori_loop(0, CAP // 16, tails, 0)

            nc = (nt + CHUNK - 1) >> 6

            def pchunk(i, carry2):
                off = i * CHUNK
                pltpu.async_copy(h_hbm.at[slist_v.at[pl.ds(off, CHUNK)]],
                                 rows_v, sem0).wait()
                pltpu.sync_copy(wlist_v.at[pl.ds(off, CHUNK)],
                                denom_s.at[didx_v.at[i]], add=True)
                for f in range(2):
                    def srow(r, carry3, f=f):
                        wspl = plsc.load_gather(
                            wlist_v, [jnp.full((16,), off + r, jnp.int32)])
                        for k in range(8):
                            half_v[r, pl.ds(k * 16, 16)] = (
                                rows_v[r, pl.ds((f * 8 + k) * 16, 16)]
                                * wspl)
                        return carry3
                    lax.fori_loop(0, CHUNK, srow, 0)
                    pltpu.sync_copy(half_v, agg_s.at[f].at[didx_v.at[i]],
                                    add=True)
                return carry2
            lax.fori_loop(0, nc, pchunk, 0)
            return carry1
        lax.fori_loop(0, STAGE // SUP, sup_body, 0)
        return carry
    lax.fori_loop(0, N_STAGES, stage_body, 0)

    plsc.subcore_barrier()

    nbase = sid * NODES_PER_TILE
    pltpu.sync_copy(denom_s.at[pl.ds(nbase, NODES_PER_TILE)], recip_v)

    def red_body(j, carry):
        acc = recip_v[pl.ds(j * 16, 16)]
        safe = jnp.where(acc > 0.0, acc, 1.0)
        recip_v[pl.ds(j * 16, 16)] = jnp.where(acc > 0.0, 1.0 / safe, 0.0)
        return carry
    lax.fori_loop(0, NODES_PER_TILE // 16, red_body, 0)

    NB = 64

    def blk_body(j, carry):
        lbase = nbase + j * NB
        for f in range(2):
            pltpu.sync_copy(agg_s.at[f, pl.ds(lbase, NB)], half_v)

            def fin_row(r, carry2, f=f):
                rsp = plsc.load_gather(
                    recip_v, [jnp.full((16,), j * NB + r, jnp.int32)])
                for k in range(8):
                    v = half_v[r, pl.ds(k * 16, 16)] * rsp
                    v = jnp.where(v > 0.0, v, jnp.exp(v) - 1.0)
                    half_v[r, pl.ds(k * 16, 16)] = v
                return carry2
            lax.fori_loop(0, NB, fin_row, 0)

            for q in range(NB // 16):
                grow = lo + lbase + q * 16

                @pl.when(grow < N_NODES)
                def _():
                    pltpu.sync_copy(
                        half_v.at[pl.ds(q * 16, 16)],
                        out_hbm.at[pl.ds(grow, 16), pl.ds(f * 128, 128)])
        return carry
    lax.fori_loop(0, NODES_PER_TILE // NB, blk_body, 0)


@functools.partial(
    pl.kernel,
    out_type=jax.ShapeDtypeStruct((N_NODES, DIM), jnp.float32),
    mesh=plsc.VectorSubcoreMesh(core_axis_name="c", subcore_axis_name="s"),
    scratch_types=[
        pltpu.VMEM((E_PER_TILE,), jnp.float32),
        pltpu.VMEM((STAGE // 128, 128), jnp.int32),
        pltpu.VMEM((STAGE // 128, 128), jnp.int32),
        pltpu.VMEM((CAP,), jnp.int32),
        pltpu.VMEM((CAP,), jnp.int32),
        pltpu.VMEM((CAP,), jnp.float32),
        pltpu.VMEM((IDXROWS, CHUNK), jnp.int32),
        pltpu.VMEM((CHUNK, DIM), jnp.float32),
        pltpu.VMEM((CHUNK, 128), jnp.float32),
        pltpu.VMEM((NODES_PER_TILE,), jnp.float32),
        pltpu.VMEM_SHARED((2, HALF, 128), jnp.float32),
        pltpu.VMEM_SHARED((HALF,), jnp.float32),
        pltpu.SemaphoreType.DMA,
    ],
    compiler_params=pltpu.CompilerParams(needs_layout_passes=False),
)
def _edge_kernel(h_hbm, a_hbm, b_hbm, src2d, dst2d, out_hbm, *scratch):
    _sc_body(h_hbm, a_hbm, b_hbm, src2d, dst2d, out_hbm, *scratch)


def kernel(x, edge_index, W, att):
    att1 = att[:DIM]
    att2 = att[DIM:]
    attm = jnp.pad(jnp.stack([att1, att2], axis=1), ((0, 0), (0, 6)))
    h, ab = _project(x, W, attm)
    a = ab[:, 0]
    b = ab[:, 1]
    src = edge_index[0].astype(jnp.int32)
    dst = edge_index[1].astype(jnp.int32)
    src2d = jnp.pad(src, (0, EP - N_EDGES)).reshape(-1, 128)
    dst2d = jnp.pad(dst, (0, EP - N_EDGES)).reshape(-1, 128)
    return _edge_kernel(h, a, b, src2d, dst2d)


# speedup vs baseline: 1.9892x; 1.9892x over previous
import functools

import jax
import jax.numpy as jnp
from jax import lax
from jax.experimental import pallas as pl
from jax.experimental.pallas import tpu as pltpu
from jax.experimental.pallas import tpu_sc as plsc

N_NODES = 10000
N_EDGES = 160000
DIM = 256
FH = 128

N_BLK = 1024
NP = 10240
NODES_PER_TILE = 640

CHUNK = 128
GRP = 8
N_GRPS = 10
N_CHUNKS = GRP * N_GRPS
E_PER_TILE = CHUNK * N_CHUNKS
EP = 16 * E_PER_TILE


def _mm_body(x_ref, w_ref, attm_ref, h0_ref, h1_ref, ab_ref):
    h = jnp.dot(x_ref[...], w_ref[...], preferred_element_type=jnp.float32)
    h0_ref[...] = h[:, :FH]
    h1_ref[...] = h[:, FH:]
    ab_ref[...] = jnp.dot(h, attm_ref[...], preferred_element_type=jnp.float32)


def _project(x, W, attm):
    xp = jnp.pad(x, ((0, NP - N_NODES), (0, 0)))
    grid = NP // N_BLK
    return pl.pallas_call(
        _mm_body,
        grid=(grid,),
        in_specs=[
            pl.BlockSpec((N_BLK, DIM), lambda i: (i, 0)),
            pl.BlockSpec((DIM, DIM), lambda i: (0, 0)),
            pl.BlockSpec((DIM, 8), lambda i: (0, 0)),
        ],
        out_specs=[
            pl.BlockSpec((N_BLK, FH), lambda i: (i, 0)),
            pl.BlockSpec((N_BLK, FH), lambda i: (i, 0)),
            pl.BlockSpec((N_BLK, 8), lambda i: (i, 0)),
        ],
        out_shape=[
            jax.ShapeDtypeStruct((NP, FH), jnp.float32),
            jax.ShapeDtypeStruct((NP, FH), jnp.float32),
            jax.ShapeDtypeStruct((NP, 8), jnp.float32),
        ],
    )(xp, W, attm)


def _sc_body(h0, h1, a_hbm, b_hbm, src2d, dst2d, out_hbm,
             wtab_v, src_v, dst_v, rows0_v, rows1_v, recip_v,
             agg_s, denom_s, sem0, sem1):
    cid = lax.axis_index("c")
    sid = lax.axis_index("s")
    zeros16 = jnp.zeros((16,), jnp.float32)

    def stage_group(g):
        row0 = sid * N_CHUNKS + g * GRP
        gp = g & 1
        pltpu.sync_copy(src2d.at[pl.ds(row0, GRP)], src_v.at[gp])
        pltpu.sync_copy(dst2d.at[pl.ds(row0, GRP)], dst_v.at[gp])

    pltpu.sync_copy(a_hbm, wtab_v)

    def prea_g(g, carry):
        stage_group(g)

        def prea_c(c, carry2):
            gc = g * GRP + c
            for k in range(8):
                s16 = src_v[g & 1, c, pl.ds(k * 16, 16)]
                rows0_v[gc, pl.ds(k * 16, 16)] = plsc.load_gather(
                    wtab_v, [s16])
            return carry2
        lax.fori_loop(0, GRP, prea_c, 0)
        return carry
    lax.fori_loop(0, N_GRPS, prea_g, 0)

    pltpu.sync_copy(b_hbm, wtab_v)

    def preb_g(g, carry):
        stage_group(g)

        def preb_c(c, carry2):
            gc = g * GRP + c
            base = sid * E_PER_TILE + gc * CHUNK
            for k in range(8):
                d16 = dst_v[g & 1, c, pl.ds(k * 16, 16)]
                z = rows0_v[gc, pl.ds(k * 16, 16)] + plsc.load_gather(
                    wtab_v, [d16])
                z = jnp.where(z >= 0.0, z, 0.2 * z)
                wv = jnp.exp(z)
                gid = base + k * 16 + jnp.arange(16, dtype=jnp.int32)
                wv = jnp.where(gid < N_EDGES, wv, 0.0)
                rows0_v[gc, pl.ds(k * 16, 16)] = wv
            return carry2
        lax.fori_loop(0, GRP, preb_c, 0)
        return carry
    lax.fori_loop(0, N_GRPS, preb_g, 0)

    def wcopy(i, carry):
        wtab_v[pl.ds(i * 16, 16)] = rows0_v[i >> 3,
                                            pl.ds((i & 7) * 16, 16)]
        return carry
    lax.fori_loop(0, E_PER_TILE // 16, wcopy, 0)

    def zero_rows(r, carry):
        for f in range(8):
            rows0_v[r, pl.ds(f * 16, 16)] = zeros16
        return carry
    lax.fori_loop(0, CHUNK, zero_rows, 0)
    for j in range(NODES_PER_TILE // CHUNK):
        pltpu.sync_copy(
            rows0_v,
            agg_s.at[pl.ds(sid * NODES_PER_TILE + j * CHUNK, CHUNK)])

    def zero_recip(i, carry):
        recip_v[pl.ds(i * 16, 16)] = zeros16
        return carry
    lax.fori_loop(0, NODES_PER_TILE // 16, zero_recip, 0)
    pltpu.sync_copy(recip_v, denom_s.at[pl.ds(sid * NODES_PER_TILE,
                                              NODES_PER_TILE)])
    plsc.subcore_barrier()

    def edge_phase(h_ref):
        def issue(c, rows, sem):
            gp = (c >> 3) & 1
            cig = c & 7
            pltpu.async_copy(h_ref.at[src_v.at[gp, cig]], rows, sem)

        def wait(c, rows, sem):
            gp = (c >> 3) & 1
            cig = c & 7
            pltpu.make_async_copy(h_ref.at[src_v.at[gp, cig]], rows,
                                  sem).wait()

        def process(c, rows):
            gp = (c >> 3) & 1
            cig = c & 7

            def scale4(rr, carry3):
                for dr in range(4):
                    r = rr * 4 + dr
                    wspl = plsc.load_gather(
                        wtab_v, [jnp.full((16,), c * CHUNK + r, jnp.int32)])
                    for f in range(8):
                        rows[r, pl.ds(f * 16, 16)] = (
                            rows[r, pl.ds(f * 16, 16)] * wspl)
                return carry3
            lax.fori_loop(0, CHUNK // 4, scale4, 0)
            pltpu.sync_copy(wtab_v.at[pl.ds(c * CHUNK, CHUNK)],
                            denom_s.at[dst_v.at[gp, cig]], add=True)
            pltpu.sync_copy(rows, agg_s.at[dst_v.at[gp, cig]], add=True)

        stage_group(0)
        issue(0, rows0_v, sem0)

        def pair_body(cc, carry):
            g = cc >> 2

            @pl.when(((cc & 3) == 3) & (g < N_GRPS - 1))
            def _():
                stage_group(g + 1)

            ca = 2 * cc
            wait(ca, rows0_v, sem0)
            issue(ca + 1, rows1_v, sem1)
            process(ca, rows0_v)
            wait(ca + 1, rows1_v, sem1)

            @pl.when(cc < N_CHUNKS // 2 - 1)
            def _():
                issue(ca + 2, rows0_v, sem0)
            process(ca + 1, rows1_v)
            return carry
        lax.fori_loop(0, N_CHUNKS // 2, pair_body, 0)

    @pl.when(cid == 0)
    def _():
        edge_phase(h0)

    @pl.when(cid == 1)
    def _():
        edge_phase(h1)

    plsc.subcore_barrier()

    nbase = sid * NODES_PER_TILE
    pltpu.sync_copy(denom_s.at[pl.ds(nbase, NODES_PER_TILE)], recip_v)

    def red_body(j, carry):
        acc = recip_v[pl.ds(j * 16, 16)]
        safe = jnp.where(acc > 0.0, acc, 1.0)
        recip_v[pl.ds(j * 16, 16)] = jnp.where(acc > 0.0, 1.0 / safe, 0.0)
        return carry
    lax.fori_loop(0, NODES_PER_TILE // 16, red_body, 0)

    def finalize(col0):
        def blk_body(j, carry):
            rbase = nbase + j * CHUNK
            pltpu.sync_copy(agg_s.at[pl.ds(rbase, CHUNK)], rows0_v)

            def fin_row(r, carry2):
                rsp = plsc.load_gather(
                    recip_v, [jnp.full((16,), j * CHUNK + r, jnp.int32)])
                for f in range(8):
                    v = rows0_v[r, pl.ds(f * 16, 16)] * rsp
                    v = jnp.where(v > 0.0, v, jnp.exp(v) - 1.0)
                    rows0_v[r, pl.ds(f * 16, 16)] = v
                return carry2
            lax.fori_loop(0, CHUNK, fin_row, 0)

            for q in range(8):
                rb = rbase + q * 16

                @pl.when(rb < N_NODES)
                def _():
                    pltpu.sync_copy(
                        rows0_v.at[pl.ds(q * 16, 16)],
                        out_hbm.at[pl.ds(rb, 16), pl.ds(col0, FH)])
            return carry
        lax.fori_loop(0, NODES_PER_TILE // CHUNK, blk_body, 0)

    @pl.when(cid == 0)
    def _():
        finalize(0)

    @pl.when(cid == 1)
    def _():
        finalize(FH)


@functools.partial(
    pl.kernel,
    out_type=jax.ShapeDtypeStruct((N_NODES, DIM), jnp.float32),
    mesh=plsc.VectorSubcoreMesh(core_axis_name="c", subcore_axis_name="s"),
    scratch_types=[
        pltpu.VMEM((E_PER_TILE,), jnp.float32),
        pltpu.VMEM((2, GRP, CHUNK), jnp.int32),
        pltpu.VMEM((2, GRP, CHUNK), jnp.int32),
        pltpu.VMEM((CHUNK, FH), jnp.float32),
        pltpu.VMEM((CHUNK, FH), jnp.float32),
        pltpu.VMEM((NODES_PER_TILE,), jnp.float32),
        pltpu.VMEM_SHARED((NP, FH), jnp.float32),
        pltpu.VMEM_SHARED((NP,), jnp.float32),
        pltpu.SemaphoreType.DMA,
        pltpu.SemaphoreType.DMA,
    ],
    compiler_params=pltpu.CompilerParams(needs_layout_passes=False),
)
def _edge_kernel(h0, h1, a_hbm, b_hbm, src2d, dst2d, out_hbm, *scratch):
    _sc_body(h0, h1, a_hbm, b_hbm, src2d, dst2d, out_hbm, *scratch)


def kernel(x, edge_index, W, att):
    att1 = att[:DIM]
    att2 = att[DIM:]
    attm = jnp.pad(jnp.stack([att1, att2], axis=1), ((0, 0), (0, 6)))
    h0, h1, ab = _project(x, W, attm)
    a = ab[:, 0]
    b = ab[:, 1]
    src = edge_index[0].astype(jnp.int32)
    dst = edge_index[1].astype(jnp.int32)
    src2d = jnp.pad(src, (0, EP - N_EDGES)).reshape(-1, CHUNK)
    dst2d = jnp.pad(dst, (0, EP - N_EDGES)).reshape(-1, CHUNK)
    return _edge_kernel(h0, h1, a, b, src2d, dst2d)


# speedup vs baseline: 2.0085x; 1.0097x over previous
import functools

import jax
import jax.numpy as jnp
from jax import lax
from jax.experimental import pallas as pl
from jax.experimental.pallas import tpu as pltpu
from jax.experimental.pallas import tpu_sc as plsc

N_NODES = 10000
N_EDGES = 160000
DIM = 256
FH = 128

N_BLK = 1024
NP = 10240
NODES_PER_TILE = 640

CHUNK = 128
GRP = 8
N_GRPS = 10
N_CHUNKS = GRP * N_GRPS
E_PER_TILE = CHUNK * N_CHUNKS
EP = 16 * E_PER_TILE


def _mm_body(x_ref, w_ref, attm_ref, h0_ref, h1_ref, ab_ref):
    h = jnp.dot(x_ref[...], w_ref[...], preferred_element_type=jnp.float32)
    h0_ref[...] = h[:, :FH]
    h1_ref[...] = h[:, FH:]
    ab_ref[...] = jnp.dot(h, attm_ref[...], preferred_element_type=jnp.float32)


def _project(x, W, attm):
    xp = jnp.pad(x, ((0, NP - N_NODES), (0, 0)))
    grid = NP // N_BLK
    return pl.pallas_call(
        _mm_body,
        grid=(grid,),
        in_specs=[
            pl.BlockSpec((N_BLK, DIM), lambda i: (i, 0)),
            pl.BlockSpec((DIM, DIM), lambda i: (0, 0)),
            pl.BlockSpec((DIM, 8), lambda i: (0, 0)),
        ],
        out_specs=[
            pl.BlockSpec((N_BLK, FH), lambda i: (i, 0)),
            pl.BlockSpec((N_BLK, FH), lambda i: (i, 0)),
            pl.BlockSpec((N_BLK, 8), lambda i: (i, 0)),
        ],
        out_shape=[
            jax.ShapeDtypeStruct((NP, FH), jnp.float32),
            jax.ShapeDtypeStruct((NP, FH), jnp.float32),
            jax.ShapeDtypeStruct((NP, 8), jnp.float32),
        ],
    )(xp, W, attm)


def _sc_body(h0, h1, a_hbm, b_hbm, src2d, dst2d, out_hbm,
             wtab_v, src_v, dst_v, rows0_v, rows1_v, recip_v,
             agg_s, denom_s, sem0, sem1, semw0, semw1):
    cid = lax.axis_index("c")
    sid = lax.axis_index("s")
    zeros16 = jnp.zeros((16,), jnp.float32)

    def stage_group(g):
        row0 = sid * N_CHUNKS + g * GRP
        gp = g & 1
        pltpu.sync_copy(src2d.at[pl.ds(row0, GRP)], src_v.at[gp])
        pltpu.sync_copy(dst2d.at[pl.ds(row0, GRP)], dst_v.at[gp])

    pltpu.sync_copy(a_hbm, wtab_v)

    def prea_g(g, carry):
        stage_group(g)

        def prea_c(c, carry2):
            gc = g * GRP + c
            for k in range(8):
                s16 = src_v[g & 1, c, pl.ds(k * 16, 16)]
                rows0_v[gc, pl.ds(k * 16, 16)] = plsc.load_gather(
                    wtab_v, [s16])
            return carry2
        lax.fori_loop(0, GRP, prea_c, 0)
        return carry
    lax.fori_loop(0, N_GRPS, prea_g, 0)

    pltpu.sync_copy(b_hbm, wtab_v)

    def preb_g(g, carry):
        stage_group(g)

        def preb_c(c, carry2):
            gc = g * GRP + c
            base = sid * E_PER_TILE + gc * CHUNK
            for k in range(8):
                d16 = dst_v[g & 1, c, pl.ds(k * 16, 16)]
                z = rows0_v[gc, pl.ds(k * 16, 16)] + plsc.load_gather(
                    wtab_v, [d16])
                z = jnp.where(z >= 0.0, z, 0.2 * z)
                wv = jnp.exp(z)
                gid = base + k * 16 + jnp.arange(16, dtype=jnp.int32)
                wv = jnp.where(gid < N_EDGES, wv, 0.0)
                rows0_v[gc, pl.ds(k * 16, 16)] = wv
            return carry2
        lax.fori_loop(0, GRP, preb_c, 0)
        return carry
    lax.fori_loop(0, N_GRPS, preb_g, 0)

    def wcopy(i, carry):
        wtab_v[pl.ds(i * 16, 16)] = rows0_v[i >> 3,
                                            pl.ds((i & 7) * 16, 16)]
        return carry
    lax.fori_loop(0, E_PER_TILE // 16, wcopy, 0)

    def zero_rows(r, carry):
        for f in range(8):
            rows0_v[r, pl.ds(f * 16, 16)] = zeros16
        return carry
    lax.fori_loop(0, CHUNK, zero_rows, 0)
    for j in range(NODES_PER_TILE // CHUNK):
        pltpu.sync_copy(
            rows0_v,
            agg_s.at[pl.ds(sid * NODES_PER_TILE + j * CHUNK, CHUNK)])

    def zero_recip(i, carry):
        recip_v[pl.ds(i * 16, 16)] = zeros16
        return carry
    lax.fori_loop(0, NODES_PER_TILE // 16, zero_recip, 0)
    pltpu.sync_copy(recip_v, denom_s.at[pl.ds(sid * NODES_PER_TILE,
                                              NODES_PER_TILE)])
    plsc.subcore_barrier()

    def edge_phase(h_ref):
        def issue(c, rows, sem):
            gp = (c >> 3) & 1
            cig = c & 7
            pltpu.async_copy(h_ref.at[src_v.at[gp, cig]], rows, sem)

        def wait(c, rows, sem):
            gp = (c >> 3) & 1
            cig = c & 7
            pltpu.make_async_copy(h_ref.at[src_v.at[gp, cig]], rows,
                                  sem).wait()

        def issue_w(c, semw):
            gp = (c >> 3) & 1
            cig = c & 7
            pltpu.async_copy(wtab_v.at[pl.ds(c * CHUNK, CHUNK)],
                             denom_s.at[dst_v.at[gp, cig]], semw, add=True)

        def wait_w(c, semw):
            gp = (c >> 3) & 1
            cig = c & 7
            pltpu.make_async_copy(wtab_v.at[pl.ds(c * CHUNK, CHUNK)],
                                  denom_s.at[dst_v.at[gp, cig]],
                                  semw).wait()

        def process(c, rows):
            gp = (c >> 3) & 1
            cig = c & 7

            def scale4(rr, carry3):
                for dr in range(4):
                    r = rr * 4 + dr
                    wspl = plsc.load_gather(
                        wtab_v, [jnp.full((16,), c * CHUNK + r, jnp.int32)])
                    for f in range(8):
                        rows[r, pl.ds(f * 16, 16)] = (
                            rows[r, pl.ds(f * 16, 16)] * wspl)
                return carry3
            lax.fori_loop(0, CHUNK // 4, scale4, 0)
            pltpu.sync_copy(rows, agg_s.at[dst_v.at[gp, cig]], add=True)

        stage_group(0)
        issue(0, rows0_v, sem0)

        def pair_body(cc, carry):
            g = cc >> 2

            @pl.when(((cc & 3) == 3) & (g < N_GRPS - 1))
            def _():
                stage_group(g + 1)

            ca = 2 * cc

            @pl.when(cc > 0)
            def _():
                wait_w(ca - 2, semw0)
                wait_w(ca - 1, semw1)
            issue_w(ca, semw0)
            issue_w(ca + 1, semw1)
            wait(ca, rows0_v, sem0)
            issue(ca + 1, rows1_v, sem1)
            process(ca, rows0_v)
            wait(ca + 1, rows1_v, sem1)

            @pl.when(cc < N_CHUNKS // 2 - 1)
            def _():
                issue(ca + 2, rows0_v, sem0)
            process(ca + 1, rows1_v)
            return carry
        lax.fori_loop(0, N_CHUNKS // 2, pair_body, 0)
        wait_w(N_CHUNKS - 2, semw0)
        wait_w(N_CHUNKS - 1, semw1)

    @pl.when(cid == 0)
    def _():
        edge_phase(h0)

    @pl.when(cid == 1)
    def _():
        edge_phase(h1)

    plsc.subcore_barrier()

    nbase = sid * NODES_PER_TILE
    pltpu.sync_copy(denom_s.at[pl.ds(nbase, NODES_PER_TILE)], recip_v)

    def red_body(j, carry):
        acc = recip_v[pl.ds(j * 16, 16)]
        safe = jnp.where(acc > 0.0, acc, 1.0)
        recip_v[pl.ds(j * 16, 16)] = jnp.where(acc > 0.0, 1.0 / safe, 0.0)
        return carry
    lax.fori_loop(0, NODES_PER_TILE // 16, red_body, 0)

    def finalize(col0):
        def blk_body(j, carry):
            rbase = nbase + j * CHUNK
            pltpu.sync_copy(agg_s.at[pl.ds(rbase, CHUNK)], rows0_v)

            def fin_row(r, carry2):
                rsp = plsc.load_gather(
                    recip_v, [jnp.full((16,), j * CHUNK + r, jnp.int32)])
                for f in range(8):
                    v = rows0_v[r, pl.ds(f * 16, 16)] * rsp
                    v = jnp.where(v > 0.0, v, jnp.exp(v) - 1.0)
                    rows0_v[r, pl.ds(f * 16, 16)] = v
                return carry2
            lax.fori_loop(0, CHUNK, fin_row, 0)

            for q in range(8):
                rb = rbase + q * 16

                @pl.when(rb < N_NODES)
                def _():
                    pltpu.sync_copy(
                        rows0_v.at[pl.ds(q * 16, 16)],
                        out_hbm.at[pl.ds(rb, 16), pl.ds(col0, FH)])
            return carry
        lax.fori_loop(0, NODES_PER_TILE // CHUNK, blk_body, 0)

    @pl.when(cid == 0)
    def _():
        finalize(0)

    @pl.when(cid == 1)
    def _():
        finalize(FH)


@functools.partial(
    pl.kernel,
    out_type=jax.ShapeDtypeStruct((N_NODES, DIM), jnp.float32),
    mesh=plsc.VectorSubcoreMesh(core_axis_name="c", subcore_axis_name="s"),
    scratch_types=[
        pltpu.VMEM((E_PER_TILE,), jnp.float32),
        pltpu.VMEM((2, GRP, CHUNK), jnp.int32),
        pltpu.VMEM((2, GRP, CHUNK), jnp.int32),
        pltpu.VMEM((CHUNK, FH), jnp.float32),
        pltpu.VMEM((CHUNK, FH), jnp.float32),
        pltpu.VMEM((NODES_PER_TILE,), jnp.float32),
        pltpu.VMEM_SHARED((NP, FH), jnp.float32),
        pltpu.VMEM_SHARED((NP,), jnp.float32),
        pltpu.SemaphoreType.DMA,
        pltpu.SemaphoreType.DMA,
        pltpu.SemaphoreType.DMA,
        pltpu.SemaphoreType.DMA,
    ],
    compiler_params=pltpu.CompilerParams(needs_layout_passes=False),
)
def _edge_kernel(h0, h1, a_hbm, b_hbm, src2d, dst2d, out_hbm, *scratch):
    _sc_body(h0, h1, a_hbm, b_hbm, src2d, dst2d, out_hbm, *scratch)


def kernel(x, edge_index, W, att):
    att1 = att[:DIM]
    att2 = att[DIM:]
    attm = jnp.pad(jnp.stack([att1, att2], axis=1), ((0, 0), (0, 6)))
    h0, h1, ab = _project(x, W, attm)
    a = ab[:, 0]
    b = ab[:, 1]
    src = edge_index[0].astype(jnp.int32)
    dst = edge_index[1].astype(jnp.int32)
    src2d = jnp.pad(src, (0, EP - N_EDGES)).reshape(-1, CHUNK)
    dst2d = jnp.pad(dst, (0, EP - N_EDGES)).reshape(-1, CHUNK)
    return _edge_kernel(h0, h1, a, b, src2d, dst2d)


# speedup vs baseline: 2.0223x; 1.0068x over previous
import functools

import jax
import jax.numpy as jnp
from jax import lax
from jax.experimental import pallas as pl
from jax.experimental.pallas import tpu as pltpu
from jax.experimental.pallas import tpu_sc as plsc

N_NODES = 10000
N_EDGES = 160000
DIM = 256
FH = 128

N_BLK = 1024
NP = 10240
NODES_PER_TILE = 640

CHUNK = 128
GRP = 8
N_GRPS = 10
N_CHUNKS = GRP * N_GRPS
E_PER_TILE = CHUNK * N_CHUNKS
EP = 16 * E_PER_TILE


def _mm_body(x_ref, w_ref, attm_ref, h0_ref, h1_ref, ab_ref):
    h = jnp.dot(x_ref[...], w_ref[...], preferred_element_type=jnp.float32)
    h0_ref[...] = h[:, :FH]
    h1_ref[...] = h[:, FH:]
    ab_ref[...] = jnp.dot(h, attm_ref[...], preferred_element_type=jnp.float32)


def _project(x, W, attm):
    xp = jnp.pad(x, ((0, NP - N_NODES), (0, 0)))
    grid = NP // N_BLK
    return pl.pallas_call(
        _mm_body,
        grid=(grid,),
        in_specs=[
            pl.BlockSpec((N_BLK, DIM), lambda i: (i, 0)),
            pl.BlockSpec((DIM, DIM), lambda i: (0, 0)),
            pl.BlockSpec((DIM, 8), lambda i: (0, 0)),
        ],
        out_specs=[
            pl.BlockSpec((N_BLK, FH), lambda i: (i, 0)),
            pl.BlockSpec((N_BLK, FH), lambda i: (i, 0)),
            pl.BlockSpec((N_BLK, 8), lambda i: (i, 0)),
        ],
        out_shape=[
            jax.ShapeDtypeStruct((NP, FH), jnp.float32),
            jax.ShapeDtypeStruct((NP, FH), jnp.float32),
            jax.ShapeDtypeStruct((NP, 8), jnp.float32),
        ],
    )(xp, W, attm)


def _sc_body(h0, h1, a_hbm, b_hbm, src2d, dst2d, out_hbm,
             wtab_v, src_v, dst_v, rows0_v, rows1_v, recip_v,
             agg_s, denom_s, sem0, sem1, semw0, semw1, semr0, semr1):
    cid = lax.axis_index("c")
    sid = lax.axis_index("s")
    zeros16 = jnp.zeros((16,), jnp.float32)

    def stage_group(g):
        row0 = sid * N_CHUNKS + g * GRP
        gp = g & 1
        pltpu.sync_copy(src2d.at[pl.ds(row0, GRP)], src_v.at[gp])
        pltpu.sync_copy(dst2d.at[pl.ds(row0, GRP)], dst_v.at[gp])

    pltpu.sync_copy(a_hbm, wtab_v)

    def prea_g(g, carry):
        stage_group(g)

        def prea_c(c, carry2):
            gc = g * GRP + c
            for k in range(8):
                s16 = src_v[g & 1, c, pl.ds(k * 16, 16)]
                rows0_v[gc, pl.ds(k * 16, 16)] = plsc.load_gather(
                    wtab_v, [s16])
            return carry2
        lax.fori_loop(0, GRP, prea_c, 0)
        return carry
    lax.fori_loop(0, N_GRPS, prea_g, 0)

    pltpu.sync_copy(b_hbm, wtab_v)

    def preb_g(g, carry):
        stage_group(g)

        def preb_c(c, carry2):
            gc = g * GRP + c
            base = sid * E_PER_TILE + gc * CHUNK
            for k in range(8):
                d16 = dst_v[g & 1, c, pl.ds(k * 16, 16)]
                z = rows0_v[gc, pl.ds(k * 16, 16)] + plsc.load_gather(
                    wtab_v, [d16])
                z = jnp.where(z >= 0.0, z, 0.2 * z)
                wv = jnp.exp(z)
                gid = base + k * 16 + jnp.arange(16, dtype=jnp.int32)
                wv = jnp.where(gid < N_EDGES, wv, 0.0)
                rows0_v[gc, pl.ds(k * 16, 16)] = wv
            return carry2
        lax.fori_loop(0, GRP, preb_c, 0)
        return carry
    lax.fori_loop(0, N_GRPS, preb_g, 0)

    def wcopy(i, carry):
        wtab_v[pl.ds(i * 16, 16)] = rows0_v[i >> 3,
                                            pl.ds((i & 7) * 16, 16)]
        return carry
    lax.fori_loop(0, E_PER_TILE // 16, wcopy, 0)

    def zero_rows(r, carry):
        for f in range(8):
            rows0_v[r, pl.ds(f * 16, 16)] = zeros16
        return carry
    lax.fori_loop(0, CHUNK, zero_rows, 0)
    for j in range(NODES_PER_TILE // CHUNK):
        pltpu.sync_copy(
            rows0_v,
            agg_s.at[pl.ds(sid * NODES_PER_TILE + j * CHUNK, CHUNK)])

    def zero_recip(i, carry):
        recip_v[pl.ds(i * 16, 16)] = zeros16
        return carry
    lax.fori_loop(0, NODES_PER_TILE // 16, zero_recip, 0)
    pltpu.sync_copy(recip_v, denom_s.at[pl.ds(sid * NODES_PER_TILE,
                                              NODES_PER_TILE)])
    plsc.subcore_barrier()

    def edge_phase(h_ref):
        def issue(c, rows, sem):
            gp = (c >> 3) & 1
            cig = c & 7
            pltpu.async_copy(h_ref.at[src_v.at[gp, cig]], rows, sem)

        def wait(c, rows, sem):
            gp = (c >> 3) & 1
            cig = c & 7
            pltpu.make_async_copy(h_ref.at[src_v.at[gp, cig]], rows,
                                  sem).wait()

        def issue_w(c, semw):
            gp = (c >> 3) & 1
            cig = c & 7
            pltpu.async_copy(wtab_v.at[pl.ds(c * CHUNK, CHUNK)],
                             denom_s.at[dst_v.at[gp, cig]], semw, add=True)

        def wait_w(c, semw):
            gp = (c >> 3) & 1
            cig = c & 7
            pltpu.make_async_copy(wtab_v.at[pl.ds(c * CHUNK, CHUNK)],
                                  denom_s.at[dst_v.at[gp, cig]],
                                  semw).wait()

        def process(c, rows):
            gp = (c >> 3) & 1
            cig = c & 7

            def scale4(rr, carry3):
                for dr in range(4):
                    r = rr * 4 + dr
                    wspl = plsc.load_gather(
                        wtab_v, [jnp.full((16,), c * CHUNK + r, jnp.int32)])
                    for f in range(8):
                        rows[r, pl.ds(f * 16, 16)] = (
                            rows[r, pl.ds(f * 16, 16)] * wspl)
                return carry3
            lax.fori_loop(0, CHUNK // 4, scale4, 0)

        def issue_r(c, rows, semr):
            gp = (c >> 3) & 1
            cig = c & 7
            pltpu.async_copy(rows, agg_s.at[dst_v.at[gp, cig]], semr,
                             add=True)

        def wait_r(c, rows, semr):
            gp = (c >> 3) & 1
            cig = c & 7
            pltpu.make_async_copy(rows, agg_s.at[dst_v.at[gp, cig]],
                                  semr).wait()

        stage_group(0)
        issue(0, rows0_v, sem0)

        def pair_body(cc, carry):
            g = cc >> 2

            @pl.when(((cc & 3) == 3) & (g < N_GRPS - 1))
            def _():
                stage_group(g + 1)

            ca = 2 * cc

            @pl.when(cc > 0)
            def _():
                wait_w(ca - 2, semw0)
                wait_w(ca - 1, semw1)
            issue_w(ca, semw0)
            issue_w(ca + 1, semw1)
            wait(ca, rows0_v, sem0)

            @pl.when(cc > 0)
            def _():
                wait_r(ca - 1, rows1_v, semr1)
            issue(ca + 1, rows1_v, sem1)
            process(ca, rows0_v)
            issue_r(ca, rows0_v, semr0)
            wait(ca + 1, rows1_v, sem1)
            wait_r(ca, rows0_v, semr0)

            @pl.when(cc < N_CHUNKS // 2 - 1)
            def _():
                issue(ca + 2, rows0_v, sem0)
            process(ca + 1, rows1_v)
            issue_r(ca + 1, rows1_v, semr1)
            return carry
        lax.fori_loop(0, N_CHUNKS // 2, pair_body, 0)
        wait_w(N_CHUNKS - 2, semw0)
        wait_w(N_CHUNKS - 1, semw1)
        wait_r(N_CHUNKS - 1, rows1_v, semr1)

    @pl.when(cid == 0)
    def _():
        edge_phase(h0)

    @pl.when(cid == 1)
    def _():
        edge_phase(h1)

    plsc.subcore_barrier()

    nbase = sid * NODES_PER_TILE
    pltpu.sync_copy(denom_s.at[pl.ds(nbase, NODES_PER_TILE)], recip_v)

    def red_body(j, carry):
        acc = recip_v[pl.ds(j * 16, 16)]
        safe = jnp.where(acc > 0.0, acc, 1.0)
        recip_v[pl.ds(j * 16, 16)] = jnp.where(acc > 0.0, 1.0 / safe, 0.0)
        return carry
    lax.fori_loop(0, NODES_PER_TILE // 16, red_body, 0)

    def finalize(col0):
        def blk_body(j, carry):
            rbase = nbase + j * CHUNK
            pltpu.sync_copy(agg_s.at[pl.ds(rbase, CHUNK)], rows0_v)

            def fin_row(r, carry2):
                rsp = plsc.load_gather(
                    recip_v, [jnp.full((16,), j * CHUNK + r, jnp.int32)])
                for f in range(8):
                    v = rows0_v[r, pl.ds(f * 16, 16)] * rsp
                    v = jnp.where(v > 0.0, v, jnp.exp(v) - 1.0)
                    rows0_v[r, pl.ds(f * 16, 16)] = v
                return carry2
            lax.fori_loop(0, CHUNK, fin_row, 0)

            for q in range(8):
                rb = rbase + q * 16

                @pl.when(rb < N_NODES)
                def _():
                    pltpu.sync_copy(
                        rows0_v.at[pl.ds(q * 16, 16)],
                        out_hbm.at[pl.ds(rb, 16), pl.ds(col0, FH)])
            return carry
        lax.fori_loop(0, NODES_PER_TILE // CHUNK, blk_body, 0)

    @pl.when(cid == 0)
    def _():
        finalize(0)

    @pl.when(cid == 1)
    def _():
        finalize(FH)


@functools.partial(
    pl.kernel,
    out_type=jax.ShapeDtypeStruct((N_NODES, DIM), jnp.float32),
    mesh=plsc.VectorSubcoreMesh(core_axis_name="c", subcore_axis_name="s"),
    scratch_types=[
        pltpu.VMEM((E_PER_TILE,), jnp.float32),
        pltpu.VMEM((2, GRP, CHUNK), jnp.int32),
        pltpu.VMEM((2, GRP, CHUNK), jnp.int32),
        pltpu.VMEM((CHUNK, FH), jnp.float32),
        pltpu.VMEM((CHUNK, FH), jnp.float32),
        pltpu.VMEM((NODES_PER_TILE,), jnp.float32),
        pltpu.VMEM_SHARED((NP, FH), jnp.float32),
        pltpu.VMEM_SHARED((NP,), jnp.float32),
        pltpu.SemaphoreType.DMA,
        pltpu.SemaphoreType.DMA,
        pltpu.SemaphoreType.DMA,
        pltpu.SemaphoreType.DMA,
        pltpu.SemaphoreType.DMA,
        pltpu.SemaphoreType.DMA,
    ],
    compiler_params=pltpu.CompilerParams(needs_layout_passes=False),
)
def _edge_kernel(h0, h1, a_hbm, b_hbm, src2d, dst2d, out_hbm, *scratch):
    _sc_body(h0, h1, a_hbm, b_hbm, src2d, dst2d, out_hbm, *scratch)


def kernel(x, edge_index, W, att):
    att1 = att[:DIM]
    att2 = att[DIM:]
    attm = jnp.pad(jnp.stack([att1, att2], axis=1), ((0, 0), (0, 6)))
    h0, h1, ab = _project(x, W, attm)
    a = ab[:, 0]
    b = ab[:, 1]
    src = edge_index[0].astype(jnp.int32)
    dst = edge_index[1].astype(jnp.int32)
    src2d = jnp.pad(src, (0, EP - N_EDGES)).reshape(-1, CHUNK)
    dst2d = jnp.pad(dst, (0, EP - N_EDGES)).reshape(-1, CHUNK)
    return _edge_kernel(h0, h1, a, b, src2d, dst2d)
